# baseline (speedup 1.0000x reference)
import jax
import jax.numpy as jnp
from jax import lax
from jax.experimental import pallas as pl
from jax.experimental.pallas import tpu as pltpu
from jax.experimental.pallas import tpu_sc as plsc

VOCAB = 1000000
LSEQ = 200
D = 64
BATCH = 4096

NC = 2
NS = 16
NW = NC * NS

WTILES = BATCH // 128
NBUF = 2

VBLOCKS = VOCAB // 128
VTAIL = VOCAB - VBLOCKS * 128
BPW = (VBLOCKS + NW - 1) // NW


def _repack_body(tokT_hbm, tail_hbm, out_hbm, in0, in1, ob0, ob1,
                 oc0, oc1, isem0, isem1, osem0, osem1):
    ibuf = (in0, in1)
    obuf = (ob0, ob1)
    obuf2 = (oc0, oc1)
    isem = (isem0, isem1)
    osem = (osem0, osem1)

    w = lax.axis_index("s") * NC + lax.axis_index("c")
    iota = lax.iota(jnp.int32, 16)
    rvecs = [iota + (vg * 16) for vg in range(8)]

    def blk_of(i):
        return w * BPW + i

    def fetch(b, i):
        @pl.when((i < BPW) & (blk_of(i) < VBLOCKS))
        def _():
            off = pl.multiple_of(blk_of(i) * 128, 128)
            pltpu.async_copy(
                tokT_hbm.at[:, pl.ds(off, 128)], ibuf[b], isem[b])

    for b in range(2):
        fetch(b, b)

    @pl.loop(0, (BPW + 1) // 2)
    def _grp(t):
        for b in range(2):
            i = t * 2 + b

            @pl.when((i < BPW) & (blk_of(i) < VBLOCKS))
            def _():
                blk = blk_of(i)
                pltpu.make_async_copy(tokT_hbm.at[:, pl.ds(0, 128)],
                                      ibuf[b], isem[b]).wait()

                @pl.when(t > 0)
                def _():
                    pltpu.make_async_copy(
                        obuf2[b], out_hbm.at[pl.ds(0, 128)], osem[b]).wait()

                @pl.loop(0, D, unroll=2)
                def _f(f):
                    fv = jnp.full((16,), f, jnp.int32)
                    vals = [ibuf[b][f, pl.ds(vg * 16, 16)]
                            for vg in range(8)]
                    for vg in range(8):
                        plsc.store_scatter(obuf[b], [rvecs[vg], fv],
                                           vals[vg])

                @pl.loop(0, 128, unroll=4)
                def _c(r):
                    for eg in range(8):
                        sl = pl.ds(eg * 16, 16)
                        obuf2[b][r, sl] = obuf[b][r, sl]

                pltpu.async_copy(
                    obuf2[b],
                    out_hbm.at[pl.ds(pl.multiple_of(blk * 128, 128), 128)],
                    osem[b])
                fetch(b, i + 2)

    for b in range(2):
        pltpu.make_async_copy(obuf2[b], out_hbm.at[pl.ds(0, 128)],
                              osem[b]).wait()

    @pl.when(w == NW - 1)
    def _tail():
        pltpu.sync_copy(tail_hbm, ibuf[0].at[pl.ds(0, 32), pl.ds(0, 128)])

        @pl.loop(0, VTAIL)
        def _v(v):
            for e in range(4):
                obuf[0][v, pl.ds(e * 16, 16)] = (
                    ibuf[0][v // 2, pl.ds((v % 2) * 64 + e * 16, 16)])

        pltpu.sync_copy(obuf[0].at[pl.ds(0, VTAIL), pl.ds(0, 128)],
                        out_hbm.at[pl.ds(VBLOCKS * 128, VTAIL)])


def _sc_body(tok_hbm, xi_hbm, pos_hbm, out_hbm,
             xall, g0, g1, o0, o1, oc0, oc1, pos_v,
             gsem0, gsem1, ssem0, ssem1):
    gbuf = (g0, g1)
    obuf = (o0, o1)
    obuf2 = (oc0, oc1)
    gsem = (gsem0, gsem1)
    ssem = (ssem0, ssem1)

    w = lax.axis_index("s") * NC + lax.axis_index("c")

    pltpu.sync_copy(pos_hbm, pos_v)
    pltpu.sync_copy(xi_hbm.at[w], xall)

    def fetch(b, l):
        pltpu.async_copy(tok_hbm.at[xall.at[l]], gbuf[b], gsem[b])

    for b in range(NBUF):
        fetch(b, b)

    iota = lax.iota(jnp.int32, 16)

    @pl.loop(0, LSEQ // NBUF)
    def _grp(t):
        for b in range(NBUF):
            l = t * NBUF + b
            pltpu.make_async_copy(tok_hbm.at[pl.ds(0, 128)], gbuf[b],
                                  gsem[b]).wait()

            @pl.when(t > 0)
            def _():
                pltpu.make_async_copy(
                    obuf2[b], out_hbm.at[0, :, 0], ssem[b]).wait()

            pr = l // 2
            pc = (l % 2) * 64

            posv = [pos_v[pr, pl.ds(pc + e * 16, 16)] for e in range(4)]
            gv = iota >> 3
            sv = iota & 7

            for e in range(4):
                gvec = gv + (e * 2)
                pse = posv[e]

                @pl.loop(0, 128, unroll=4)
                def _j(j):
                    val = gbuf[b][j, pl.ds(e * 16, 16)] + pse
                    plsc.store_scatter(
                        obuf[b], [gvec, sv, jnp.full((16,), j, jnp.int32)],
                        val)

            @pl.loop(0, 8)
            def _cg(g):
                for cs in range(8):
                    for eg in range(8):
                        sl = pl.ds(eg * 16, 16)
                        obuf2[b][g, cs, sl] = obuf[b][g, cs, sl]

            pltpu.async_copy(obuf2[b], out_hbm.at[l, :, w], ssem[b])

            @pl.when(l + NBUF < LSEQ)
            def _():
                fetch(b, l + NBUF)

    for b in range(NBUF):
        pltpu.make_async_copy(obuf2[b], out_hbm.at[0, :, 0], ssem[b]).wait()


@jax.jit
def _sc_embed(tokT, tail2, xi3, pos2):
    mesh = plsc.VectorSubcoreMesh(core_axis_name="c", subcore_axis_name="s")
    params = pltpu.CompilerParams(use_tc_tiling_on_sc=True,
                                  needs_layout_passes=False)
    params_lin = pltpu.CompilerParams(use_tc_tiling_on_sc=False,
                                      needs_layout_passes=False)
    repack = pl.kernel(
        _repack_body,
        out_type=jax.ShapeDtypeStruct((VOCAB, 128), jnp.float32),
        mesh=mesh,
        scratch_types=[
            pltpu.VMEM((D, 128), jnp.float32),
            pltpu.VMEM((D, 128), jnp.float32),
            pltpu.VMEM((128, 129), jnp.float32),
            pltpu.VMEM((128, 129), jnp.float32),
            pltpu.VMEM((128, 128), jnp.float32),
            pltpu.VMEM((128, 128), jnp.float32),
            pltpu.SemaphoreType.DMA,
            pltpu.SemaphoreType.DMA,
            pltpu.SemaphoreType.DMA,
            pltpu.SemaphoreType.DMA,
        ],
        compiler_params=params,
    )
    tok2 = repack(tokT, tail2)
    lookup = pl.kernel(
        _sc_body,
        out_type=jax.ShapeDtypeStruct((LSEQ, 8, WTILES, 8, 128), jnp.float32),
        mesh=mesh,
        scratch_types=[
            pltpu.VMEM((LSEQ, 128), jnp.int32),
            pltpu.VMEM((128, 128), jnp.float32),
            pltpu.VMEM((128, 128), jnp.float32),
            pltpu.VMEM((8, 8, 129), jnp.float32),
            pltpu.VMEM((8, 8, 129), jnp.float32),
            pltpu.VMEM((8, 8, 128), jnp.float32),
            pltpu.VMEM((8, 8, 128), jnp.float32),
            pltpu.VMEM((100, 128), jnp.float32),
            pltpu.SemaphoreType.DMA,
            pltpu.SemaphoreType.DMA,
            pltpu.SemaphoreType.DMA,
            pltpu.SemaphoreType.DMA,
        ],
        compiler_params=params_lin,
    )
    return lookup(tok2, xi3, pos2)


def kernel(x, token_table, pos_table):
    xi3 = x.astype(jnp.int32).T.reshape(LSEQ, WTILES, 128).transpose(1, 0, 2)
    pos2 = pos_table.reshape(100, 128)
    tail2 = token_table[VBLOCKS * 128:].reshape(32, 128)
    out5 = _sc_embed(token_table.T, tail2, xi3, pos2)
    return out5.transpose(2, 4, 0, 1, 3).reshape(BATCH, LSEQ, D)


# speedup vs baseline: 2.5784x; 2.5784x over previous
import jax
import jax.numpy as jnp
from jax import lax
from jax.experimental import pallas as pl
from jax.experimental.pallas import tpu as pltpu
from jax.experimental.pallas import tpu_sc as plsc

VOCAB = 1000000
LSEQ = 200
D = 64
BATCH = 4096

NC = 2
NS = 16
NW = NC * NS

WTILES = BATCH // 128
NBUF = 4


def _sc_body(tok_hbm, xi_hbm, pos_hbm, out_hbm,
             xall, g0, g1, g2, g3, o0, o1, o2, o3, pos_v,
             gsem0, gsem1, gsem2, gsem3, ssem0, ssem1, ssem2, ssem3):
    gbuf = (g0, g1, g2, g3)
    obuf = (o0, o1, o2, o3)
    gsem = (gsem0, gsem1, gsem2, gsem3)
    ssem = (ssem0, ssem1, ssem2, ssem3)

    w = lax.axis_index("s") * NC + lax.axis_index("c")

    pltpu.sync_copy(pos_hbm, pos_v)
    pltpu.sync_copy(xi_hbm.at[:, w], xall)

    def fetch(b, l):
        pltpu.async_copy(tok_hbm.at[xall.at[l]], gbuf[b], gsem[b])

    for b in range(NBUF):
        fetch(b, b)

    iota = lax.iota(jnp.int32, 16)

    @pl.loop(0, LSEQ // NBUF)
    def _grp(t):
        for b in range(NBUF):
            l = t * NBUF + b
            pltpu.make_async_copy(tok_hbm.at[pl.ds(0, 128)], gbuf[b],
                                  gsem[b]).wait()

            @pl.when(t > 0)
            def _():
                pltpu.make_async_copy(
                    obuf[b].at[:, :, pl.ds(0, 128)],
                    out_hbm.at[0, :, 0], ssem[b]).wait()

            pr = l // 2
            pc = (l % 2) * 64

            posv = [pos_v[pr, pl.ds(pc + e * 16, 16)] for e in range(4)]
            gv = iota >> 3
            sv = iota & 7

            for e in range(4):
                gvec = gv + (e * 2)
                pse = posv[e]

                @pl.loop(0, 128, unroll=4)
                def _j(j):
                    val = gbuf[b][j, pl.ds(e * 16, 16)] + pse
                    plsc.store_scatter(
                        obuf[b], [gvec, sv, jnp.full((16,), j, jnp.int32)],
                        val)

            pltpu.async_copy(obuf[b].at[:, :, pl.ds(0, 128)],
                             out_hbm.at[l, :, w], ssem[b])

            @pl.when(l + NBUF < LSEQ)
            def _():
                fetch(b, l + NBUF)

    for b in range(NBUF):
        pltpu.make_async_copy(obuf[b].at[:, :, pl.ds(0, 128)],
                              out_hbm.at[0, :, 0], ssem[b]).wait()


@jax.jit
def _sc_embed(tok, xi3, pos2):
    mesh = plsc.VectorSubcoreMesh(core_axis_name="c", subcore_axis_name="s")
    fn = pl.kernel(
        _sc_body,
        out_type=jax.ShapeDtypeStruct((LSEQ, 8, WTILES, 8, 128), jnp.float32),
        mesh=mesh,
        scratch_types=[
            pltpu.VMEM((LSEQ, 128), jnp.int32),
            pltpu.VMEM((128, D), jnp.float32),
            pltpu.VMEM((128, D), jnp.float32),
            pltpu.VMEM((128, D), jnp.float32),
            pltpu.VMEM((128, D), jnp.float32),
            pltpu.VMEM((8, 8, 129), jnp.float32),
            pltpu.VMEM((8, 8, 129), jnp.float32),
            pltpu.VMEM((8, 8, 129), jnp.float32),
            pltpu.VMEM((8, 8, 129), jnp.float32),
            pltpu.VMEM((100, 128), jnp.float32),
            pltpu.SemaphoreType.DMA,
            pltpu.SemaphoreType.DMA,
            pltpu.SemaphoreType.DMA,
            pltpu.SemaphoreType.DMA,
            pltpu.SemaphoreType.DMA,
            pltpu.SemaphoreType.DMA,
            pltpu.SemaphoreType.DMA,
            pltpu.SemaphoreType.DMA,
        ],
        compiler_params=pltpu.CompilerParams(use_tc_tiling_on_sc=False,
                                             needs_layout_passes=False),
    )
    return fn(tok, xi3, pos2)


def kernel(x, token_table, pos_table):
    xi3 = x.astype(jnp.int32).T.reshape(LSEQ, WTILES, 128)
    pos2 = pos_table.reshape(100, 128)
    out5 = _sc_embed(token_table, xi3, pos2)
    return out5.transpose(2, 4, 0, 1, 3).reshape(BATCH, LSEQ, D)
